# baseline (speedup 1.0000x reference)
import dataclasses
import functools

import jax
import jax.numpy as jnp
from jax import lax
from jax.experimental import pallas as pl
from jax.experimental.pallas import tpu as pltpu
from jax.experimental.pallas import tpu_sc as plsc

_ALPHA = 12.0
_C_BONDS, _C_AROMA, _C_CHARGE, _C_ELEM = 5, 2, 13, 54
_B, _N = 64, 128

_KEYS = (
    (1832780943, 270669613),
    (64467757, 2916123636),
    (2465931498, 255383827),
    (3134548294, 894150801),
)

_TINY = float(jnp.finfo(jnp.float32).tiny)

_ROT = ((13, 15, 26, 6), (17, 29, 16, 24))


def _threefry_bits(idx, k0, k1):
    k0 = int(k0)
    k1 = int(k1)
    ks2 = (k0 ^ k1 ^ 0x1BD11BDA) & 0xFFFFFFFF
    ks = (k0, k1, ks2)
    x0 = jnp.full(idx.shape, jnp.uint32(k0), jnp.uint32)
    x1 = idx + jnp.uint32(k1)
    for g in range(1, 6):
        for r in _ROT[(g - 1) % 2]:
            x0 = x0 + x1
            x1 = (x1 << r) | (x1 >> (32 - r))
            x1 = x1 ^ x0
        x0 = x0 + jnp.uint32(ks[g % 3])
        x1 = x1 + jnp.uint32((ks[(g + 1) % 3] + g) & 0xFFFFFFFF)
    return x0 ^ x1


def _gumbel(idx, k0, k1):
    bits = _threefry_bits(idx, k0, k1)
    flo = pltpu.bitcast(
        (bits >> 9) | jnp.uint32(0x3F800000), jnp.float32) - jnp.float32(1.0)
    u = jnp.maximum(jnp.float32(_TINY), flo + jnp.float32(_TINY))
    return -jnp.log(-jnp.log(u))


def _sample_classes(pred, curr, init, cf, idx_row, C, k0, k1):
    at, bt, dte, inv1mt, invt = (cf(j) for j in range(5))
    m = pred[0]
    for c in range(1, C):
        m = jnp.maximum(m, pred[c])
    e = [jnp.exp(pred[c] - m) for c in range(C)]
    s = e[0]
    for c in range(1, C):
        s = s + e[c]
    best_val = None
    best_idx = None
    for c in range(C):
        p1c = e[c] / s
        ptc = (curr == c).astype(jnp.float32)
        p0c = (init == c).astype(jnp.float32)
        fwd = inv1mt * (p1c - ptc)
        bwd = invt * (p0c - ptc)
        u = at * fwd - bt * bwd
        prob = jnp.maximum(u * dte + ptc, jnp.float32(1e-10))
        idx_elem = (idx_row * C + c).astype(jnp.uint32)
        score = jnp.log(prob) + _gumbel(idx_elem, k0, k1)
        if c == 0:
            best_val = score
            best_idx = jnp.zeros_like(curr)
        else:
            gt = score > best_val
            best_val = jnp.where(gt, score, best_val)
            best_idx = jnp.where(gt, c, best_idx)
    return best_idx


_WQ = 64


def _bonds_body(pred_ref, curr_ref, init_ref, coef_ref, out_ref, *, base):
    i = pl.program_id(0)
    roff = (lax.broadcasted_iota(jnp.int32, (_WQ, 128), 0) * 128
            + lax.broadcasted_iota(jnp.int32, (_WQ, 128), 1))
    idx_row = base + i * (_WQ * 128) + roff
    cf = lambda j: coef_ref[0, j:j + 1, :]
    pred = [pred_ref[c] for c in range(_C_BONDS)]
    out_ref[0] = _sample_classes(pred, curr_ref[0], init_ref[0], cf, idx_row,
                                 _C_BONDS, *_KEYS[0])


def _sc_log(u):
    bits = plsc.bitcast(u, jnp.int32)
    e = ((bits >> 23) & 0xFF) - 127
    m = plsc.bitcast((bits & 0x007FFFFF) | 0x3F800000, jnp.float32)
    big = m > jnp.float32(1.4142135381698608)
    m = jnp.where(big, m * jnp.float32(0.5), m)
    e = jnp.where(big, e + 1, e).astype(jnp.float32)
    x = m - jnp.float32(1.0)
    z = x * x
    p = jnp.float32(7.0376836292e-2)
    for coef_ in (-1.1514610310e-1, 1.5410766671e-1, -1.2420140846e-1,
                  1.4249322787e-1, -1.6668057665e-1, 2.0000714765e-1,
                  -2.4999993993e-1, 3.3333331174e-1):
        p = p * x + jnp.float32(coef_)
    y = x * z * p
    y = y + e * jnp.float32(-2.12194440e-4)
    y = y - jnp.float32(0.5) * z
    return x + y + e * jnp.float32(0.693359375)


def _sc_threefry_gumbel(idx, k0, k1):
    k0 = int(k0)
    k1 = int(k1)
    ks2 = (k0 ^ k1 ^ 0x1BD11BDA) & 0xFFFFFFFF
    ks = (k0, k1, ks2)
    x0 = jnp.zeros((16,), jnp.uint32) + jnp.uint32(k0)
    x1 = lax.convert_element_type(idx, jnp.uint32) + jnp.uint32(k1)
    for g in range(1, 6):
        for r in _ROT[(g - 1) % 2]:
            x0 = x0 + x1
            x1 = (x1 << r) | (x1 >> (32 - r))
            x1 = x1 ^ x0
        x0 = x0 + jnp.uint32(ks[g % 3])
        x1 = x1 + jnp.uint32((ks[(g + 1) % 3] + g) & 0xFFFFFFFF)
    bits = x0 ^ x1
    flo = plsc.bitcast((bits >> 9) | jnp.uint32(0x3F800000), jnp.float32) - 1.0
    u = jnp.maximum(jnp.float32(_TINY), flo + jnp.float32(_TINY))
    return -_sc_log(-_sc_log(u))


_SC_ROWS = 256
_SC_U = 4


def _sc_smalls_body(pa, ca, ia, pc, cc, ic, pe, ce, ie,
                    at_h, bt_h, dte_h, i1_h, it_h, oa, oc, oe,
                    pav, pcv, pev, curv, iniv, outv,
                    atv, btv, dtev, i1v, itv,
                    m_ref, s_ref, bv_ref, bi_ref, sem):
    cid = lax.axis_index("c")
    sid = lax.axis_index("s")
    base = (cid * 16 + sid) * _SC_ROWS

    for ch, cv in ((at_h, atv), (bt_h, btv), (dte_h, dtev),
                   (i1_h, i1v), (it_h, itv)):
        pltpu.async_copy(ch.at[pl.ds(base, _SC_ROWS)], cv, sem).wait()

    for C, ph, chh, ih, oh, predv, key in (
            (_C_AROMA, pa, ca, ia, oa, pav, _KEYS[1]),
            (_C_CHARGE, pc, cc, ic, oc, pcv, _KEYS[2]),
            (_C_ELEM, pe, ce, ie, oe, pev, _KEYS[3])):
        k0, k1 = key
        pltpu.async_copy(ph.at[pl.ds(base, _SC_ROWS), :], predv, sem).wait()
        pltpu.async_copy(chh.at[pl.ds(base, _SC_ROWS)], curv, sem).wait()
        pltpu.async_copy(ih.at[pl.ds(base, _SC_ROWS)], iniv, sem).wait()

        @pl.loop(0, _SC_ROWS // (16 * _SC_U))
        def _(g):
            r0 = g * (16 * _SC_U)
            lane = lax.iota(jnp.int32, 16)

            def rows_of(u):
                return r0 + u * 16 + lane

            def gath(u, c):
                cvec = jnp.zeros((16,), jnp.int32) + c
                return plsc.load_gather(predv, [rows_of(u), cvec])

            def sl(u):
                return pl.ds(r0 + u * 16, 16)

            def usl(u):
                return pl.ds(u * 16, 16)

            for u in range(_SC_U):
                m_ref[usl(u)] = jnp.zeros((16,), jnp.float32) - jnp.float32(
                    jnp.inf)

            @pl.loop(0, C)
            def _(c):
                for u in range(_SC_U):
                    m_ref[usl(u)] = jnp.maximum(m_ref[usl(u)], gath(u, c))

            for u in range(_SC_U):
                s_ref[usl(u)] = jnp.zeros((16,), jnp.float32)

            @pl.loop(0, C)
            def _(c):
                for u in range(_SC_U):
                    s_ref[usl(u)] = s_ref[usl(u)] + jnp.exp(
                        gath(u, c) - m_ref[usl(u)])

            for u in range(_SC_U):
                bv_ref[usl(u)] = jnp.zeros((16,), jnp.float32) - jnp.float32(
                    jnp.inf)
                bi_ref[usl(u)] = jnp.zeros((16,), jnp.int32)

            @pl.loop(0, C)
            def _(c):
                for u in range(_SC_U):
                    p1c = jnp.exp(gath(u, c) - m_ref[usl(u)]) / s_ref[usl(u)]
                    ptc = (curv[sl(u)] == c).astype(jnp.float32)
                    p0c = (iniv[sl(u)] == c).astype(jnp.float32)
                    fwd = i1v[sl(u)] * (p1c - ptc)
                    bwd = itv[sl(u)] * (p0c - ptc)
                    uu = atv[sl(u)] * fwd - btv[sl(u)] * bwd
                    prob = jnp.maximum(uu * dtev[sl(u)] + ptc,
                                       jnp.float32(1e-10))
                    idx = (base + rows_of(u)) * C + c
                    score = _sc_log(prob) + _sc_threefry_gumbel(idx, k0, k1)
                    gt = score > bv_ref[usl(u)]
                    bi_ref[usl(u)] = jnp.where(gt, c, bi_ref[usl(u)])
                    bv_ref[usl(u)] = jnp.where(gt, score, bv_ref[usl(u)])

            for u in range(_SC_U):
                outv[sl(u)] = bi_ref[usl(u)]

        pltpu.async_copy(outv, oh.at[pl.ds(base, _SC_ROWS)], sem).wait()


def kernel(curr_bonds, pred_bonds, init_bonds, curr_aromas, pred_aromas,
           init_aromas, curr_charges, pred_charges, init_charges,
           curr_element_types, pred_element_types, init_element_types, t, dt):
    B, N = _B, _N
    M = B * N * N
    Mq = M // 128
    Ms = B * N
    nq = Ms // 128

    at = 1.0 + _ALPHA * t ** 2.0 * (1.0 - t) ** 0.5
    bt = at - 1.0
    alpha_term = at * 1.0 / (1.0 - t)
    beta_term = bt * 1.0 / t
    dte = jnp.minimum(dt, 1.0 / (alpha_term + beta_term))
    inv1mt = 1.0 / (1.0 - t)
    invt = 1.0 / t
    coef = jnp.stack([at, bt, dte, inv1mt, invt], axis=0)

    coef_bonds = jnp.broadcast_to(coef.T[:, :, None], (B, 5, 128))
    coef_small = jnp.broadcast_to(coef[:, :, None], (5, B, 128))

    blk3 = lambda C: pl.BlockSpec((C, _WQ, 128), lambda i: (0, i, 0))
    pred_b = pred_bonds.reshape(M, _C_BONDS).T.reshape(_C_BONDS, Mq, 128)
    curr_b = curr_bonds.reshape(1, Mq, 128)
    init_b = init_bonds.reshape(1, Mq, 128)

    out_bonds = pl.pallas_call(
        functools.partial(_bonds_body, base=0),
        grid=(Mq // _WQ,),
        in_specs=[
            blk3(_C_BONDS),
            blk3(1),
            blk3(1),
            pl.BlockSpec((1, 5, 128), lambda i: (i // 2, 0, 0)),
        ],
        out_specs=blk3(1),
        out_shape=jax.ShapeDtypeStruct((1, Mq, 128), jnp.int32),
    )(pred_b, curr_b, init_b, coef_bonds).reshape(B, N, N)

    coef_rows = jnp.broadcast_to(coef[:, :, None], (5, B, N)).reshape(5, Ms)

    mesh = plsc.VectorSubcoreMesh(core_axis_name="c", subcore_axis_name="s")
    out_small = jax.ShapeDtypeStruct((Ms,), jnp.int32)
    cp = pltpu.CompilerParams()
    if "needs_layout_passes" in pltpu.CompilerParams.__dataclass_fields__:
        cp = dataclasses.replace(cp, needs_layout_passes=False)
    sck = pl.kernel(
        _sc_smalls_body,
        out_type=[out_small] * 3,
        mesh=mesh,
        compiler_params=cp,
        scratch_types=[
            pltpu.VMEM((_SC_ROWS, _C_AROMA), jnp.float32),
            pltpu.VMEM((_SC_ROWS, _C_CHARGE), jnp.float32),
            pltpu.VMEM((_SC_ROWS, _C_ELEM), jnp.float32),
            pltpu.VMEM((_SC_ROWS,), jnp.int32),
            pltpu.VMEM((_SC_ROWS,), jnp.int32),
            pltpu.VMEM((_SC_ROWS,), jnp.int32),
            pltpu.VMEM((_SC_ROWS,), jnp.float32),
            pltpu.VMEM((_SC_ROWS,), jnp.float32),
            pltpu.VMEM((_SC_ROWS,), jnp.float32),
            pltpu.VMEM((_SC_ROWS,), jnp.float32),
            pltpu.VMEM((_SC_ROWS,), jnp.float32),
            pltpu.VMEM((16 * _SC_U,), jnp.float32),
            pltpu.VMEM((16 * _SC_U,), jnp.float32),
            pltpu.VMEM((16 * _SC_U,), jnp.float32),
            pltpu.VMEM((16 * _SC_U,), jnp.int32),
            pltpu.SemaphoreType.DMA,
        ],
    )
    out_a, out_c, out_e = sck(
        pred_aromas.reshape(Ms, _C_AROMA), curr_aromas.reshape(Ms),
        init_aromas.reshape(Ms),
        pred_charges.reshape(Ms, _C_CHARGE), curr_charges.reshape(Ms),
        init_charges.reshape(Ms),
        pred_element_types.reshape(Ms, _C_ELEM),
        curr_element_types.reshape(Ms), init_element_types.reshape(Ms),
        coef_rows[0], coef_rows[1], coef_rows[2], coef_rows[3], coef_rows[4])

    return (out_bonds, out_a.reshape(B, N),
            out_c.reshape(B, N), out_e.reshape(B, N))


# speedup vs baseline: 1.1260x; 1.1260x over previous
import dataclasses
import functools

import jax
import jax.numpy as jnp
from jax import lax
from jax.experimental import pallas as pl
from jax.experimental.pallas import tpu as pltpu
from jax.experimental.pallas import tpu_sc as plsc

_ALPHA = 12.0
_C_BONDS, _C_AROMA, _C_CHARGE, _C_ELEM = 5, 2, 13, 54
_B, _N = 64, 128

_KEYS = (
    (1832780943, 270669613),
    (64467757, 2916123636),
    (2465931498, 255383827),
    (3134548294, 894150801),
)

_TINY = float(jnp.finfo(jnp.float32).tiny)

_ROT = ((13, 15, 26, 6), (17, 29, 16, 24))


def _threefry_bits(idx, k0, k1):
    k0 = int(k0)
    k1 = int(k1)
    ks2 = (k0 ^ k1 ^ 0x1BD11BDA) & 0xFFFFFFFF
    ks = (k0, k1, ks2)
    x0 = jnp.full(idx.shape, jnp.uint32(k0), jnp.uint32)
    x1 = idx + jnp.uint32(k1)
    for g in range(1, 6):
        for r in _ROT[(g - 1) % 2]:
            x0 = x0 + x1
            x1 = (x1 << r) | (x1 >> (32 - r))
            x1 = x1 ^ x0
        x0 = x0 + jnp.uint32(ks[g % 3])
        x1 = x1 + jnp.uint32((ks[(g + 1) % 3] + g) & 0xFFFFFFFF)
    return x0 ^ x1


def _gumbel(idx, k0, k1):
    bits = _threefry_bits(idx, k0, k1)
    flo = pltpu.bitcast(
        (bits >> 9) | jnp.uint32(0x3F800000), jnp.float32) - jnp.float32(1.0)
    u = jnp.maximum(jnp.float32(_TINY), flo + jnp.float32(_TINY))
    return -jnp.log(-jnp.log(u))


def _sample_classes(pred, curr, init, cf, idx_row, C, k0, k1):
    at, bt, dte, inv1mt, invt = (cf(j) for j in range(5))
    m = pred[0]
    for c in range(1, C):
        m = jnp.maximum(m, pred[c])
    e = [jnp.exp(pred[c] - m) for c in range(C)]
    s = e[0]
    for c in range(1, C):
        s = s + e[c]
    best_val = None
    best_idx = None
    for c in range(C):
        p1c = e[c] / s
        ptc = (curr == c).astype(jnp.float32)
        p0c = (init == c).astype(jnp.float32)
        fwd = inv1mt * (p1c - ptc)
        bwd = invt * (p0c - ptc)
        u = at * fwd - bt * bwd
        prob = jnp.maximum(u * dte + ptc, jnp.float32(1e-10))
        idx_elem = (idx_row * C + c).astype(jnp.uint32)
        score = jnp.log(prob) + _gumbel(idx_elem, k0, k1)
        if c == 0:
            best_val = score
            best_idx = jnp.zeros_like(curr)
        else:
            gt = score > best_val
            best_val = jnp.where(gt, score, best_val)
            best_idx = jnp.where(gt, c, best_idx)
    return best_idx


_WQ = 64


def _bonds_body(pred_ref, curr_ref, init_ref, coef_ref, out_ref, *, base):
    i = pl.program_id(0)
    roff = (lax.broadcasted_iota(jnp.int32, (_WQ, 128), 0) * 128
            + lax.broadcasted_iota(jnp.int32, (_WQ, 128), 1))
    idx_row = base + i * (_WQ * 128) + roff
    cf = lambda j: coef_ref[0, j:j + 1, :]
    pred = [pred_ref[c] for c in range(_C_BONDS)]
    out_ref[0] = _sample_classes(pred, curr_ref[0], init_ref[0], cf, idx_row,
                                 _C_BONDS, *_KEYS[0])


def _sc_log(u):
    bits = plsc.bitcast(u, jnp.int32)
    e = ((bits >> 23) & 0xFF) - 127
    m = plsc.bitcast((bits & 0x007FFFFF) | 0x3F800000, jnp.float32)
    big = m > jnp.float32(1.4142135381698608)
    m = jnp.where(big, m * jnp.float32(0.5), m)
    e = jnp.where(big, e + 1, e).astype(jnp.float32)
    x = m - jnp.float32(1.0)
    z = x * x
    p = jnp.float32(7.0376836292e-2)
    for coef_ in (-1.1514610310e-1, 1.5410766671e-1, -1.2420140846e-1,
                  1.4249322787e-1, -1.6668057665e-1, 2.0000714765e-1,
                  -2.4999993993e-1, 3.3333331174e-1):
        p = p * x + jnp.float32(coef_)
    y = x * z * p
    y = y + e * jnp.float32(-2.12194440e-4)
    y = y - jnp.float32(0.5) * z
    return x + y + e * jnp.float32(0.693359375)


def _sc_threefry_gumbel(idx, k0, k1):
    k0 = int(k0)
    k1 = int(k1)
    ks2 = (k0 ^ k1 ^ 0x1BD11BDA) & 0xFFFFFFFF
    ks = (k0, k1, ks2)
    x0 = jnp.zeros((16,), jnp.uint32) + jnp.uint32(k0)
    x1 = lax.convert_element_type(idx, jnp.uint32) + jnp.uint32(k1)
    for g in range(1, 6):
        for r in _ROT[(g - 1) % 2]:
            x0 = x0 + x1
            x1 = (x1 << r) | (x1 >> (32 - r))
            x1 = x1 ^ x0
        x0 = x0 + jnp.uint32(ks[g % 3])
        x1 = x1 + jnp.uint32((ks[(g + 1) % 3] + g) & 0xFFFFFFFF)
    bits = x0 ^ x1
    flo = plsc.bitcast((bits >> 9) | jnp.uint32(0x3F800000), jnp.float32) - 1.0
    u = jnp.maximum(jnp.float32(_TINY), flo + jnp.float32(_TINY))
    return -_sc_log(-_sc_log(u))


_SC_ROWS = 256
_SC_U = 2


def _sc_smalls_body(pa, ca, ia, pc, cc, ic, pe, ce, ie,
                    at_h, bt_h, dte_h, i1_h, it_h, oa, oc, oe,
                    pav, pcv, pev, curv, iniv, outv,
                    atv, btv, dtev, i1v, itv, sem):
    cid = lax.axis_index("c")
    sid = lax.axis_index("s")
    base = (cid * 16 + sid) * _SC_ROWS

    for ch, cv in ((at_h, atv), (bt_h, btv), (dte_h, dtev),
                   (i1_h, i1v), (it_h, itv)):
        pltpu.async_copy(ch.at[pl.ds(base, _SC_ROWS)], cv, sem).wait()

    for C, ph, chh, ih, oh, predv, key in (
            (_C_AROMA, pa, ca, ia, oa, pav, _KEYS[1]),
            (_C_CHARGE, pc, cc, ic, oc, pcv, _KEYS[2]),
            (_C_ELEM, pe, ce, ie, oe, pev, _KEYS[3])):
        k0, k1 = key
        pltpu.async_copy(ph.at[pl.ds(base * C, _SC_ROWS * C)], predv,
                         sem).wait()
        pltpu.async_copy(chh.at[pl.ds(base, _SC_ROWS)], curv, sem).wait()
        pltpu.async_copy(ih.at[pl.ds(base, _SC_ROWS)], iniv, sem).wait()

        @pl.loop(0, _SC_ROWS // (16 * _SC_U))
        def _(g):
            r0 = g * (16 * _SC_U)
            lane = lax.iota(jnp.int32, 16)
            rows = [r0 + u * 16 + lane for u in range(_SC_U)]

            def gath(u, c):
                return plsc.load_gather(predv, [rows[u] * C + c])

            def sl(u):
                return pl.ds(r0 + u * 16, 16)

            ninf = jnp.zeros((16,), jnp.float32) - jnp.float32(jnp.inf)

            def p_max(c, mm):
                return tuple(jnp.maximum(mm[u], gath(u, c))
                             for u in range(_SC_U))

            m = lax.fori_loop(0, C, p_max, (ninf,) * _SC_U)

            def p_sum(c, ss):
                return tuple(ss[u] + jnp.exp(gath(u, c) - m[u])
                             for u in range(_SC_U))

            s = lax.fori_loop(0, C, p_sum,
                              (jnp.zeros((16,), jnp.float32),) * _SC_U)

            def p_score(c, carry):
                bv, bi = carry
                nbv, nbi = [], []
                for u in range(_SC_U):
                    p1c = jnp.exp(gath(u, c) - m[u]) / s[u]
                    ptc = (curv[sl(u)] == c).astype(jnp.float32)
                    p0c = (iniv[sl(u)] == c).astype(jnp.float32)
                    fwd = i1v[sl(u)] * (p1c - ptc)
                    bwd = itv[sl(u)] * (p0c - ptc)
                    uu = atv[sl(u)] * fwd - btv[sl(u)] * bwd
                    prob = jnp.maximum(uu * dtev[sl(u)] + ptc,
                                       jnp.float32(1e-10))
                    idx = (base + rows[u]) * C + c
                    score = _sc_log(prob) + _sc_threefry_gumbel(idx, k0, k1)
                    gt = score > bv[u]
                    nbi.append(jnp.where(gt, c, bi[u]))
                    nbv.append(jnp.where(gt, score, bv[u]))
                return tuple(nbv), tuple(nbi)

            _, bi = lax.fori_loop(
                0, C, p_score,
                ((ninf,) * _SC_U, (jnp.zeros((16,), jnp.int32),) * _SC_U))

            for u in range(_SC_U):
                outv[sl(u)] = bi[u]

        pltpu.async_copy(outv, oh.at[pl.ds(base, _SC_ROWS)], sem).wait()


def kernel(curr_bonds, pred_bonds, init_bonds, curr_aromas, pred_aromas,
           init_aromas, curr_charges, pred_charges, init_charges,
           curr_element_types, pred_element_types, init_element_types, t, dt):
    B, N = _B, _N
    M = B * N * N
    Mq = M // 128
    Ms = B * N
    nq = Ms // 128

    at = 1.0 + _ALPHA * t ** 2.0 * (1.0 - t) ** 0.5
    bt = at - 1.0
    alpha_term = at * 1.0 / (1.0 - t)
    beta_term = bt * 1.0 / t
    dte = jnp.minimum(dt, 1.0 / (alpha_term + beta_term))
    inv1mt = 1.0 / (1.0 - t)
    invt = 1.0 / t
    coef = jnp.stack([at, bt, dte, inv1mt, invt], axis=0)

    coef_bonds = jnp.broadcast_to(coef.T[:, :, None], (B, 5, 128))
    coef_small = jnp.broadcast_to(coef[:, :, None], (5, B, 128))

    blk3 = lambda C: pl.BlockSpec((C, _WQ, 128), lambda i: (0, i, 0))
    pred_b = pred_bonds.reshape(M, _C_BONDS).T.reshape(_C_BONDS, Mq, 128)
    curr_b = curr_bonds.reshape(1, Mq, 128)
    init_b = init_bonds.reshape(1, Mq, 128)

    out_bonds = pl.pallas_call(
        functools.partial(_bonds_body, base=0),
        grid=(Mq // _WQ,),
        in_specs=[
            blk3(_C_BONDS),
            blk3(1),
            blk3(1),
            pl.BlockSpec((1, 5, 128), lambda i: (i // 2, 0, 0)),
        ],
        out_specs=blk3(1),
        out_shape=jax.ShapeDtypeStruct((1, Mq, 128), jnp.int32),
    )(pred_b, curr_b, init_b, coef_bonds).reshape(B, N, N)

    def per_row(x):
        return jnp.broadcast_to(x[:, None], (B, N)).reshape(Ms)

    mesh = plsc.VectorSubcoreMesh(core_axis_name="c", subcore_axis_name="s")
    out_small = jax.ShapeDtypeStruct((Ms,), jnp.int32)
    cp = pltpu.CompilerParams()
    if "needs_layout_passes" in pltpu.CompilerParams.__dataclass_fields__:
        cp = dataclasses.replace(cp, needs_layout_passes=False)
    sck = pl.kernel(
        _sc_smalls_body,
        out_type=[out_small] * 3,
        mesh=mesh,
        compiler_params=cp,
        scratch_types=[
            pltpu.VMEM((_SC_ROWS * _C_AROMA,), jnp.float32),
            pltpu.VMEM((_SC_ROWS * _C_CHARGE,), jnp.float32),
            pltpu.VMEM((_SC_ROWS * _C_ELEM,), jnp.float32),
            pltpu.VMEM((_SC_ROWS,), jnp.int32),
            pltpu.VMEM((_SC_ROWS,), jnp.int32),
            pltpu.VMEM((_SC_ROWS,), jnp.int32),
            pltpu.VMEM((_SC_ROWS,), jnp.float32),
            pltpu.VMEM((_SC_ROWS,), jnp.float32),
            pltpu.VMEM((_SC_ROWS,), jnp.float32),
            pltpu.VMEM((_SC_ROWS,), jnp.float32),
            pltpu.VMEM((_SC_ROWS,), jnp.float32),
            pltpu.SemaphoreType.DMA,
        ],
    )
    out_a, out_c, out_e = sck(
        pred_aromas.reshape(Ms * _C_AROMA), curr_aromas.reshape(Ms),
        init_aromas.reshape(Ms),
        pred_charges.reshape(Ms * _C_CHARGE), curr_charges.reshape(Ms),
        init_charges.reshape(Ms),
        pred_element_types.reshape(Ms * _C_ELEM),
        curr_element_types.reshape(Ms), init_element_types.reshape(Ms),
        per_row(at), per_row(bt), per_row(dte), per_row(inv1mt), per_row(invt))

    return (out_bonds, out_a.reshape(B, N),
            out_c.reshape(B, N), out_e.reshape(B, N))


# speedup vs baseline: 1.1344x; 1.0075x over previous
import dataclasses
import functools

import jax
import jax.numpy as jnp
from jax import lax
from jax.experimental import pallas as pl
from jax.experimental.pallas import tpu as pltpu
from jax.experimental.pallas import tpu_sc as plsc

_ALPHA = 12.0
_C_BONDS, _C_AROMA, _C_CHARGE, _C_ELEM = 5, 2, 13, 54
_B, _N = 64, 128

_KEYS = (
    (1832780943, 270669613),
    (64467757, 2916123636),
    (2465931498, 255383827),
    (3134548294, 894150801),
)

_TINY = float(jnp.finfo(jnp.float32).tiny)

_ROT = ((13, 15, 26, 6), (17, 29, 16, 24))


def _threefry_bits(idx, k0, k1):
    k0 = int(k0)
    k1 = int(k1)
    ks2 = (k0 ^ k1 ^ 0x1BD11BDA) & 0xFFFFFFFF
    ks = (k0, k1, ks2)
    x0 = jnp.full(idx.shape, jnp.uint32(k0), jnp.uint32)
    x1 = idx + jnp.uint32(k1)
    for g in range(1, 6):
        for r in _ROT[(g - 1) % 2]:
            x0 = x0 + x1
            x1 = (x1 << r) | (x1 >> (32 - r))
            x1 = x1 ^ x0
        x0 = x0 + jnp.uint32(ks[g % 3])
        x1 = x1 + jnp.uint32((ks[(g + 1) % 3] + g) & 0xFFFFFFFF)
    return x0 ^ x1


def _gumbel(idx, k0, k1):
    bits = _threefry_bits(idx, k0, k1)
    flo = pltpu.bitcast(
        (bits >> 9) | jnp.uint32(0x3F800000), jnp.float32) - jnp.float32(1.0)
    u = jnp.maximum(jnp.float32(_TINY), flo + jnp.float32(_TINY))
    return -jnp.log(-jnp.log(u))


def _sample_classes(pred, curr, init, cf, idx_row, C, k0, k1):
    at, bt, dte, inv1mt, invt = (cf(j) for j in range(5))
    m = pred[0]
    for c in range(1, C):
        m = jnp.maximum(m, pred[c])
    e = [jnp.exp(pred[c] - m) for c in range(C)]
    s = e[0]
    for c in range(1, C):
        s = s + e[c]
    best_val = None
    best_idx = None
    for c in range(C):
        p1c = e[c] / s
        ptc = (curr == c).astype(jnp.float32)
        p0c = (init == c).astype(jnp.float32)
        fwd = inv1mt * (p1c - ptc)
        bwd = invt * (p0c - ptc)
        u = at * fwd - bt * bwd
        prob = jnp.maximum(u * dte + ptc, jnp.float32(1e-10))
        idx_elem = (idx_row * C + c).astype(jnp.uint32)
        score = jnp.log(prob) + _gumbel(idx_elem, k0, k1)
        if c == 0:
            best_val = score
            best_idx = jnp.zeros_like(curr)
        else:
            gt = score > best_val
            best_val = jnp.where(gt, score, best_val)
            best_idx = jnp.where(gt, c, best_idx)
    return best_idx


_WQ = 64


def _bonds_body(pred_ref, curr_ref, init_ref, coef_ref, out_ref, *, base):
    i = pl.program_id(0)
    roff = (lax.broadcasted_iota(jnp.int32, (_WQ, 128), 0) * 128
            + lax.broadcasted_iota(jnp.int32, (_WQ, 128), 1))
    idx_row = base + i * (_WQ * 128) + roff
    cf = lambda j: coef_ref[0, j:j + 1, :]
    pred = [pred_ref[c] for c in range(_C_BONDS)]
    out_ref[0] = _sample_classes(pred, curr_ref[0], init_ref[0], cf, idx_row,
                                 _C_BONDS, *_KEYS[0])


def _sc_log(u):
    bits = plsc.bitcast(u, jnp.int32)
    e = ((bits >> 23) & 0xFF) - 127
    m = plsc.bitcast((bits & 0x007FFFFF) | 0x3F800000, jnp.float32)
    big = m > jnp.float32(1.4142135381698608)
    m = jnp.where(big, m * jnp.float32(0.5), m)
    e = jnp.where(big, e + 1, e).astype(jnp.float32)
    x = m - jnp.float32(1.0)
    z = x * x
    p = jnp.float32(7.0376836292e-2)
    for coef_ in (-1.1514610310e-1, 1.5410766671e-1, -1.2420140846e-1,
                  1.4249322787e-1, -1.6668057665e-1, 2.0000714765e-1,
                  -2.4999993993e-1, 3.3333331174e-1):
        p = p * x + jnp.float32(coef_)
    y = x * z * p
    y = y + e * jnp.float32(-2.12194440e-4)
    y = y - jnp.float32(0.5) * z
    return x + y + e * jnp.float32(0.693359375)


def _sc_threefry_gumbel(idx, k0, k1):
    k0 = int(k0)
    k1 = int(k1)
    ks2 = (k0 ^ k1 ^ 0x1BD11BDA) & 0xFFFFFFFF
    ks = (k0, k1, ks2)
    x0 = jnp.zeros((16,), jnp.uint32) + jnp.uint32(k0)
    x1 = lax.convert_element_type(idx, jnp.uint32) + jnp.uint32(k1)
    for g in range(1, 6):
        for r in _ROT[(g - 1) % 2]:
            x0 = x0 + x1
            x1 = (x1 << r) | (x1 >> (32 - r))
            x1 = x1 ^ x0
        x0 = x0 + jnp.uint32(ks[g % 3])
        x1 = x1 + jnp.uint32((ks[(g + 1) % 3] + g) & 0xFFFFFFFF)
    bits = x0 ^ x1
    flo = plsc.bitcast((bits >> 9) | jnp.uint32(0x3F800000), jnp.float32) - 1.0
    u = jnp.maximum(jnp.float32(_TINY), flo + jnp.float32(_TINY))
    return -_sc_log(-_sc_log(u))


_SC_ROWS = 256
_SC_U = 2


def _sc_smalls_body(pa, ca, ia, pc, cc, ic, pe, ce, ie,
                    at_h, bt_h, dte_h, i1_h, it_h, oa, oc, oe,
                    pav, pcv, pev, curv, iniv, outv,
                    atv, btv, dtev, i1v, itv, sem):
    cid = lax.axis_index("c")
    sid = lax.axis_index("s")
    base = (cid * 16 + sid) * _SC_ROWS

    for ch, cv in ((at_h, atv), (bt_h, btv), (dte_h, dtev),
                   (i1_h, i1v), (it_h, itv)):
        pltpu.async_copy(ch.at[pl.ds(base, _SC_ROWS)], cv, sem).wait()

    for C, ph, chh, ih, oh, predv, key in (
            (_C_AROMA, pa, ca, ia, oa, pav, _KEYS[1]),
            (_C_CHARGE, pc, cc, ic, oc, pcv, _KEYS[2]),
            (_C_ELEM, pe, ce, ie, oe, pev, _KEYS[3])):
        k0, k1 = key
        pltpu.async_copy(ph.at[pl.ds(base * 128, _SC_ROWS * 128)], predv,
                         sem).wait()
        pltpu.async_copy(chh.at[pl.ds(base, _SC_ROWS)], curv, sem).wait()
        pltpu.async_copy(ih.at[pl.ds(base, _SC_ROWS)], iniv, sem).wait()

        @pl.loop(0, _SC_ROWS // (16 * _SC_U))
        def _(g):
            r0 = g * (16 * _SC_U)
            lane = lax.iota(jnp.int32, 16)
            rows = [r0 + u * 16 + lane for u in range(_SC_U)]

            def gath(u, c):
                return plsc.load_gather(predv, [rows[u] * 128 + c])

            def sl(u):
                return pl.ds(r0 + u * 16, 16)

            ninf = jnp.zeros((16,), jnp.float32) - jnp.float32(jnp.inf)

            def p_max(c, mm):
                return tuple(jnp.maximum(mm[u], gath(u, c))
                             for u in range(_SC_U))

            m = lax.fori_loop(0, C, p_max, (ninf,) * _SC_U)

            def p_sum(c, ss):
                return tuple(ss[u] + jnp.exp(gath(u, c) - m[u])
                             for u in range(_SC_U))

            s = lax.fori_loop(0, C, p_sum,
                              (jnp.zeros((16,), jnp.float32),) * _SC_U)

            def p_score(c, carry):
                bv, bi = carry
                nbv, nbi = [], []
                for u in range(_SC_U):
                    p1c = jnp.exp(gath(u, c) - m[u]) / s[u]
                    ptc = (curv[sl(u)] == c).astype(jnp.float32)
                    p0c = (iniv[sl(u)] == c).astype(jnp.float32)
                    fwd = i1v[sl(u)] * (p1c - ptc)
                    bwd = itv[sl(u)] * (p0c - ptc)
                    uu = atv[sl(u)] * fwd - btv[sl(u)] * bwd
                    prob = jnp.maximum(uu * dtev[sl(u)] + ptc,
                                       jnp.float32(1e-10))
                    idx = (base + rows[u]) * C + c
                    score = _sc_log(prob) + _sc_threefry_gumbel(idx, k0, k1)
                    gt = score > bv[u]
                    nbi.append(jnp.where(gt, c, bi[u]))
                    nbv.append(jnp.where(gt, score, bv[u]))
                return tuple(nbv), tuple(nbi)

            _, bi = lax.fori_loop(
                0, C, p_score,
                ((ninf,) * _SC_U, (jnp.zeros((16,), jnp.int32),) * _SC_U))

            for u in range(_SC_U):
                outv[sl(u)] = bi[u]

        pltpu.async_copy(outv, oh.at[pl.ds(base, _SC_ROWS)], sem).wait()


def kernel(curr_bonds, pred_bonds, init_bonds, curr_aromas, pred_aromas,
           init_aromas, curr_charges, pred_charges, init_charges,
           curr_element_types, pred_element_types, init_element_types, t, dt):
    B, N = _B, _N
    M = B * N * N
    Mq = M // 128
    Ms = B * N
    nq = Ms // 128

    at = 1.0 + _ALPHA * t ** 2.0 * (1.0 - t) ** 0.5
    bt = at - 1.0
    alpha_term = at * 1.0 / (1.0 - t)
    beta_term = bt * 1.0 / t
    dte = jnp.minimum(dt, 1.0 / (alpha_term + beta_term))
    inv1mt = 1.0 / (1.0 - t)
    invt = 1.0 / t
    coef = jnp.stack([at, bt, dte, inv1mt, invt], axis=0)

    coef_bonds = jnp.broadcast_to(coef.T[:, :, None], (B, 5, 128))
    coef_small = jnp.broadcast_to(coef[:, :, None], (5, B, 128))

    blk3 = lambda C: pl.BlockSpec((C, _WQ, 128), lambda i: (0, i, 0))
    pred_b = pred_bonds.reshape(M, _C_BONDS).T.reshape(_C_BONDS, Mq, 128)
    curr_b = curr_bonds.reshape(1, Mq, 128)
    init_b = init_bonds.reshape(1, Mq, 128)

    out_bonds = pl.pallas_call(
        functools.partial(_bonds_body, base=0),
        grid=(Mq // _WQ,),
        in_specs=[
            blk3(_C_BONDS),
            blk3(1),
            blk3(1),
            pl.BlockSpec((1, 5, 128), lambda i: (i // 2, 0, 0)),
        ],
        out_specs=blk3(1),
        out_shape=jax.ShapeDtypeStruct((1, Mq, 128), jnp.int32),
    )(pred_b, curr_b, init_b, coef_bonds).reshape(B, N, N)

    def per_row(x):
        return jnp.broadcast_to(x[:, None], (B, N)).reshape(Ms)

    mesh = plsc.VectorSubcoreMesh(core_axis_name="c", subcore_axis_name="s")
    out_small = jax.ShapeDtypeStruct((Ms,), jnp.int32)
    cp = pltpu.CompilerParams()
    if "needs_layout_passes" in pltpu.CompilerParams.__dataclass_fields__:
        cp = dataclasses.replace(cp, needs_layout_passes=False)
    sck = pl.kernel(
        _sc_smalls_body,
        out_type=[out_small] * 3,
        mesh=mesh,
        compiler_params=cp,
        scratch_types=[
            pltpu.VMEM((_SC_ROWS * 128,), jnp.float32),
            pltpu.VMEM((_SC_ROWS * 128,), jnp.float32),
            pltpu.VMEM((_SC_ROWS * 128,), jnp.float32),
            pltpu.VMEM((_SC_ROWS,), jnp.int32),
            pltpu.VMEM((_SC_ROWS,), jnp.int32),
            pltpu.VMEM((_SC_ROWS,), jnp.int32),
            pltpu.VMEM((_SC_ROWS,), jnp.float32),
            pltpu.VMEM((_SC_ROWS,), jnp.float32),
            pltpu.VMEM((_SC_ROWS,), jnp.float32),
            pltpu.VMEM((_SC_ROWS,), jnp.float32),
            pltpu.VMEM((_SC_ROWS,), jnp.float32),
            pltpu.SemaphoreType.DMA,
        ],
    )
    def pad128(pred, C):
        return jnp.pad(pred.reshape(Ms, C),
                       ((0, 0), (0, 128 - C))).reshape(Ms * 128)

    out_a, out_c, out_e = sck(
        pad128(pred_aromas, _C_AROMA), curr_aromas.reshape(Ms),
        init_aromas.reshape(Ms),
        pad128(pred_charges, _C_CHARGE), curr_charges.reshape(Ms),
        init_charges.reshape(Ms),
        pad128(pred_element_types, _C_ELEM),
        curr_element_types.reshape(Ms), init_element_types.reshape(Ms),
        per_row(at), per_row(bt), per_row(dte), per_row(inv1mt), per_row(invt))

    return (out_bonds, out_a.reshape(B, N),
            out_c.reshape(B, N), out_e.reshape(B, N))


# speedup vs baseline: 1.1363x; 1.0017x over previous
import dataclasses
import functools

import jax
import jax.numpy as jnp
from jax import lax
from jax.experimental import pallas as pl
from jax.experimental.pallas import tpu as pltpu
from jax.experimental.pallas import tpu_sc as plsc

_ALPHA = 12.0
_C_BONDS, _C_AROMA, _C_CHARGE, _C_ELEM = 5, 2, 13, 54
_B, _N = 64, 128

_KEYS = (
    (1832780943, 270669613),
    (64467757, 2916123636),
    (2465931498, 255383827),
    (3134548294, 894150801),
)

_TINY = float(jnp.finfo(jnp.float32).tiny)

_ROT = ((13, 15, 26, 6), (17, 29, 16, 24))


def _threefry_bits(idx, k0, k1):
    k0 = int(k0)
    k1 = int(k1)
    ks2 = (k0 ^ k1 ^ 0x1BD11BDA) & 0xFFFFFFFF
    ks = (k0, k1, ks2)
    x0 = jnp.full(idx.shape, jnp.uint32(k0), jnp.uint32)
    x1 = idx + jnp.uint32(k1)
    for g in range(1, 6):
        for r in _ROT[(g - 1) % 2]:
            x0 = x0 + x1
            x1 = (x1 << r) | (x1 >> (32 - r))
            x1 = x1 ^ x0
        x0 = x0 + jnp.uint32(ks[g % 3])
        x1 = x1 + jnp.uint32((ks[(g + 1) % 3] + g) & 0xFFFFFFFF)
    return x0 ^ x1


def _gumbel(idx, k0, k1):
    bits = _threefry_bits(idx, k0, k1)
    flo = pltpu.bitcast(
        (bits >> 9) | jnp.uint32(0x3F800000), jnp.float32) - jnp.float32(1.0)
    u = jnp.maximum(jnp.float32(_TINY), flo + jnp.float32(_TINY))
    return -jnp.log(-jnp.log(u))


def _sample_classes(pred, curr, init, cf, idx_row, C, k0, k1):
    at, bt, dte, inv1mt, invt = (cf(j) for j in range(5))
    m = pred[0]
    for c in range(1, C):
        m = jnp.maximum(m, pred[c])
    e = [jnp.exp(pred[c] - m) for c in range(C)]
    s = e[0]
    for c in range(1, C):
        s = s + e[c]
    best_val = None
    best_idx = None
    for c in range(C):
        p1c = e[c] / s
        ptc = (curr == c).astype(jnp.float32)
        p0c = (init == c).astype(jnp.float32)
        fwd = inv1mt * (p1c - ptc)
        bwd = invt * (p0c - ptc)
        u = at * fwd - bt * bwd
        prob = jnp.maximum(u * dte + ptc, jnp.float32(1e-10))
        idx_elem = (idx_row * C + c).astype(jnp.uint32)
        score = jnp.log(prob) + _gumbel(idx_elem, k0, k1)
        if c == 0:
            best_val = score
            best_idx = jnp.zeros_like(curr)
        else:
            gt = score > best_val
            best_val = jnp.where(gt, score, best_val)
            best_idx = jnp.where(gt, c, best_idx)
    return best_idx


_WQ = 64


def _bonds_body(pred_ref, curr_ref, init_ref, coef_ref, out_ref, *, base):
    i = pl.program_id(0)
    roff = (lax.broadcasted_iota(jnp.int32, (_WQ, 128), 0) * 128
            + lax.broadcasted_iota(jnp.int32, (_WQ, 128), 1))
    idx_row = base + i * (_WQ * 128) + roff
    cf = lambda j: coef_ref[0, j:j + 1, :]
    pred = [pred_ref[c] for c in range(_C_BONDS)]
    out_ref[0] = _sample_classes(pred, curr_ref[0], init_ref[0], cf, idx_row,
                                 _C_BONDS, *_KEYS[0])


def _sc_log(u):
    bits = plsc.bitcast(u, jnp.int32)
    e = ((bits >> 23) & 0xFF) - 127
    m = plsc.bitcast((bits & 0x007FFFFF) | 0x3F800000, jnp.float32)
    big = m > jnp.float32(1.4142135381698608)
    m = jnp.where(big, m * jnp.float32(0.5), m)
    e = jnp.where(big, e + 1, e).astype(jnp.float32)
    x = m - jnp.float32(1.0)
    z = x * x
    p = jnp.float32(7.0376836292e-2)
    for coef_ in (-1.1514610310e-1, 1.5410766671e-1, -1.2420140846e-1,
                  1.4249322787e-1, -1.6668057665e-1, 2.0000714765e-1,
                  -2.4999993993e-1, 3.3333331174e-1):
        p = p * x + jnp.float32(coef_)
    y = x * z * p
    y = y + e * jnp.float32(-2.12194440e-4)
    y = y - jnp.float32(0.5) * z
    return x + y + e * jnp.float32(0.693359375)


def _sc_threefry_gumbel(idx, k0, k1):
    k0 = int(k0)
    k1 = int(k1)
    ks2 = (k0 ^ k1 ^ 0x1BD11BDA) & 0xFFFFFFFF
    ks = (k0, k1, ks2)
    x0 = jnp.zeros((16,), jnp.uint32) + jnp.uint32(k0)
    x1 = lax.convert_element_type(idx, jnp.uint32) + jnp.uint32(k1)
    for g in range(1, 6):
        for r in _ROT[(g - 1) % 2]:
            x0 = x0 + x1
            x1 = (x1 << r) | (x1 >> (32 - r))
            x1 = x1 ^ x0
        x0 = x0 + jnp.uint32(ks[g % 3])
        x1 = x1 + jnp.uint32((ks[(g + 1) % 3] + g) & 0xFFFFFFFF)
    bits = x0 ^ x1
    flo = plsc.bitcast((bits >> 9) | jnp.uint32(0x3F800000), jnp.float32) - 1.0
    u = jnp.maximum(jnp.float32(_TINY), flo + jnp.float32(_TINY))
    return -_sc_log(-_sc_log(u))


_SC_ROWS = 256
_SC_U = 2


def _sc_smalls_body(pa, ca, ia, pc, cc, ic, pe, ce, ie,
                    at_h, bt_h, dte_h, i1_h, it_h, oa, oc, oe,
                    pav, pcv, pev, curv, iniv, outv,
                    atv, btv, dtev, i1v, itv, sem):
    cid = lax.axis_index("c")
    sid = lax.axis_index("s")
    base = (cid * 16 + sid) * _SC_ROWS

    for ch, cv in ((at_h, atv), (bt_h, btv), (dte_h, dtev),
                   (i1_h, i1v), (it_h, itv)):
        pltpu.async_copy(ch.at[pl.ds(base, _SC_ROWS)], cv, sem).wait()

    for C, ph, chh, ih, oh, predv, key in (
            (_C_AROMA, pa, ca, ia, oa, pav, _KEYS[1]),
            (_C_CHARGE, pc, cc, ic, oc, pcv, _KEYS[2]),
            (_C_ELEM, pe, ce, ie, oe, pev, _KEYS[3])):
        k0, k1 = key
        pltpu.async_copy(ph.at[pl.ds(base * 128, _SC_ROWS * 128)], predv,
                         sem).wait()
        pltpu.async_copy(chh.at[pl.ds(base, _SC_ROWS)], curv, sem).wait()
        pltpu.async_copy(ih.at[pl.ds(base, _SC_ROWS)], iniv, sem).wait()

        @pl.loop(0, _SC_ROWS // (16 * _SC_U))
        def _(g):
            r0 = g * (16 * _SC_U)
            lane = lax.iota(jnp.int32, 16)
            rows = [r0 + u * 16 + lane for u in range(_SC_U)]

            def gath(u, c):
                return plsc.load_gather(predv, [rows[u] * 128 + c])

            def sl(u):
                return pl.ds(r0 + u * 16, 16)

            ninf = jnp.zeros((16,), jnp.float32) - jnp.float32(jnp.inf)

            def p_max(c, mm):
                return tuple(jnp.maximum(mm[u], gath(u, c))
                             for u in range(_SC_U))

            m = lax.fori_loop(0, C, p_max, (ninf,) * _SC_U)

            def p_sum(c, ss):
                return tuple(ss[u] + jnp.exp(gath(u, c) - m[u])
                             for u in range(_SC_U))

            s = lax.fori_loop(0, C, p_sum,
                              (jnp.zeros((16,), jnp.float32),) * _SC_U)

            def p_score(c, carry):
                bv, bi = carry
                nbv, nbi = [], []
                for u in range(_SC_U):
                    p1c = jnp.exp(gath(u, c) - m[u]) / s[u]
                    ptc = (curv[sl(u)] == c).astype(jnp.float32)
                    p0c = (iniv[sl(u)] == c).astype(jnp.float32)
                    fwd = i1v[sl(u)] * (p1c - ptc)
                    bwd = itv[sl(u)] * (p0c - ptc)
                    uu = atv[sl(u)] * fwd - btv[sl(u)] * bwd
                    prob = jnp.maximum(uu * dtev[sl(u)] + ptc,
                                       jnp.float32(1e-10))
                    idx = (base + rows[u]) * C + c
                    score = _sc_log(prob) + _sc_threefry_gumbel(idx, k0, k1)
                    gt = score > bv[u]
                    nbi.append(jnp.where(gt, c, bi[u]))
                    nbv.append(jnp.where(gt, score, bv[u]))
                return tuple(nbv), tuple(nbi)

            _, bi = lax.fori_loop(
                0, C, p_score,
                ((ninf,) * _SC_U, (jnp.zeros((16,), jnp.int32),) * _SC_U))

            for u in range(_SC_U):
                outv[sl(u)] = bi[u]

        pltpu.async_copy(outv, oh.at[pl.ds(base, _SC_ROWS)], sem).wait()


def kernel(curr_bonds, pred_bonds, init_bonds, curr_aromas, pred_aromas,
           init_aromas, curr_charges, pred_charges, init_charges,
           curr_element_types, pred_element_types, init_element_types, t, dt):
    B, N = _B, _N
    M = B * N * N
    Mq = M // 128
    Ms = B * N
    nq = Ms // 128

    at = 1.0 + _ALPHA * t ** 2.0 * (1.0 - t) ** 0.5
    bt = at - 1.0
    alpha_term = at * 1.0 / (1.0 - t)
    beta_term = bt * 1.0 / t
    dte = jnp.minimum(dt, 1.0 / (alpha_term + beta_term))
    inv1mt = 1.0 / (1.0 - t)
    invt = 1.0 / t
    coef = jnp.stack([at, bt, dte, inv1mt, invt], axis=0)

    coef_bonds = jnp.broadcast_to(coef.T[:, :, None], (B, 5, 128))

    blk3 = lambda C: pl.BlockSpec((C, _WQ, 128), lambda i: (0, i, 0))
    pred_b = pred_bonds.reshape(M, _C_BONDS).T.reshape(_C_BONDS, Mq, 128)
    curr_b = curr_bonds.reshape(1, Mq, 128)
    init_b = init_bonds.reshape(1, Mq, 128)

    out_bonds = pl.pallas_call(
        functools.partial(_bonds_body, base=0),
        grid=(Mq // _WQ,),
        in_specs=[
            blk3(_C_BONDS),
            blk3(1),
            blk3(1),
            pl.BlockSpec((1, 5, 128), lambda i: (i // 2, 0, 0)),
        ],
        out_specs=blk3(1),
        out_shape=jax.ShapeDtypeStruct((1, Mq, 128), jnp.int32),
    )(pred_b, curr_b, init_b, coef_bonds).reshape(B, N, N)

    def per_row(x):
        return jnp.broadcast_to(x[:, None], (B, N)).reshape(Ms)

    mesh = plsc.VectorSubcoreMesh(core_axis_name="c", subcore_axis_name="s")
    out_small = jax.ShapeDtypeStruct((Ms,), jnp.int32)
    cp = pltpu.CompilerParams()
    if "needs_layout_passes" in pltpu.CompilerParams.__dataclass_fields__:
        cp = dataclasses.replace(cp, needs_layout_passes=False)
    sck = pl.kernel(
        _sc_smalls_body,
        out_type=[out_small] * 3,
        mesh=mesh,
        compiler_params=cp,
        scratch_types=[
            pltpu.VMEM((_SC_ROWS * 128,), jnp.float32),
            pltpu.VMEM((_SC_ROWS * 128,), jnp.float32),
            pltpu.VMEM((_SC_ROWS * 128,), jnp.float32),
            pltpu.VMEM((_SC_ROWS,), jnp.int32),
            pltpu.VMEM((_SC_ROWS,), jnp.int32),
            pltpu.VMEM((_SC_ROWS,), jnp.int32),
            pltpu.VMEM((_SC_ROWS,), jnp.float32),
            pltpu.VMEM((_SC_ROWS,), jnp.float32),
            pltpu.VMEM((_SC_ROWS,), jnp.float32),
            pltpu.VMEM((_SC_ROWS,), jnp.float32),
            pltpu.VMEM((_SC_ROWS,), jnp.float32),
            pltpu.SemaphoreType.DMA,
        ],
    )
    def pad128(pred, C):
        return jnp.pad(pred.reshape(Ms, C),
                       ((0, 0), (0, 128 - C))).reshape(Ms * 128)

    out_a, out_c, out_e = sck(
        pad128(pred_aromas, _C_AROMA), curr_aromas.reshape(Ms),
        init_aromas.reshape(Ms),
        pad128(pred_charges, _C_CHARGE), curr_charges.reshape(Ms),
        init_charges.reshape(Ms),
        pad128(pred_element_types, _C_ELEM),
        curr_element_types.reshape(Ms), init_element_types.reshape(Ms),
        per_row(at), per_row(bt), per_row(dte), per_row(inv1mt), per_row(invt))

    return (out_bonds, out_a.reshape(B, N),
            out_c.reshape(B, N), out_e.reshape(B, N))
